# Initial kernel scaffold; baseline (speedup 1.0000x reference)
import jax
import jax.numpy as jnp
from jax.experimental import pallas as pl


def kernel(x, edge_index, W0, b0, W1a, b1a, g1, be1, W1b, b1b, W2a, b2a, g2, be2, W2b, b2b):
    raise NotImplementedError("write your pallas kernel here")


# speedup vs baseline: 3.3123x; 3.3123x over previous
import functools

import jax
import jax.numpy as jnp
from jax import lax
from jax.experimental import pallas as pl
from jax.experimental.pallas import tpu as pltpu
from jax.experimental.pallas import tpu_sc as plsc

N = 10000
E = 320000
D = 128
H = 128

NC = 2
NS = 16
NW = NC * NS

NP = 10240
ROWS_PER_TILE = NP // NS
E_PER_W = E // NW
CH = 80
E_PER_W_PAD = 10240
CHUNKS = E_PER_W_PAD // CH

_sc_mesh = plsc.VectorSubcoreMesh(core_axis_name="c", subcore_axis_name="s")


@functools.partial(
    pl.kernel,
    out_type=jax.ShapeDtypeStruct((NC, NP, D), jnp.float32),
    mesh=_sc_mesh,
    scratch_types=[
        pltpu.VMEM((CHUNKS, CH), jnp.int32),
        pltpu.VMEM((CHUNKS, CH), jnp.int32),
        pltpu.VMEM((CH, D), jnp.float32),
        pltpu.VMEM_SHARED((NP, D), jnp.float32),
        pltpu.SemaphoreType.DMA,
    ],
)
def _sc_segment_sum(h_hbm, src_hbm, dst_hbm, zeros_hbm, out_hbm,
                    src_v, dst_v, rows_v, acc, sem):
    c = lax.axis_index("c")
    s = lax.axis_index("s")
    wid = s * NC + c
    r0 = pl.multiple_of(s * ROWS_PER_TILE, ROWS_PER_TILE)
    pltpu.sync_copy(zeros_hbm.at[pl.ds(r0, ROWS_PER_TILE)],
                    acc.at[pl.ds(r0, ROWS_PER_TILE)])
    pltpu.sync_copy(src_hbm.at[wid], src_v)
    pltpu.sync_copy(dst_hbm.at[wid], dst_v)
    plsc.subcore_barrier()

    def body(j, carry):
        pltpu.async_copy(h_hbm.at[src_v.at[j]], rows_v, sem).wait()
        pltpu.sync_copy(rows_v, acc.at[dst_v.at[j]], add=True)
        return carry

    lax.fori_loop(0, CHUNKS, body, 0)
    plsc.subcore_barrier()
    pltpu.sync_copy(acc.at[pl.ds(r0, ROWS_PER_TILE)],
                    out_hbm.at[c, pl.ds(r0, ROWS_PER_TILE)])


def _tc_linear_body(x_ref, w_ref, b_ref, o_ref):
    o_ref[...] = (jnp.dot(x_ref[...], w_ref[...],
                          preferred_element_type=jnp.float32) + b_ref[...])


def _tc_linear(x, W, b):
    return pl.pallas_call(
        _tc_linear_body,
        out_shape=jax.ShapeDtypeStruct((N, H), jnp.float32),
    )(x, W, b.reshape(1, H))


def _tc_tail_body(relu_out, h_ref, p_ref, wa_ref, ba_ref, g_ref, be_ref,
                  wb_ref, bb_ref, o_ref):
    out = h_ref[...] + p_ref[0, :N] + p_ref[1, :N]
    t = (jnp.dot(out, wa_ref[...], preferred_element_type=jnp.float32)
         + ba_ref[...])
    m = jnp.mean(t, axis=0, keepdims=True)
    v = jnp.mean((t - m) ** 2, axis=0, keepdims=True)
    t = (t - m) * lax.rsqrt(v + 1e-5) * g_ref[...] + be_ref[...]
    t = jnp.maximum(t, 0.0)
    l = (jnp.dot(t, wb_ref[...], preferred_element_type=jnp.float32)
         + bb_ref[...])
    cm = jnp.mean(l, axis=0, keepdims=True)
    rn = jnp.sqrt(1e-6 + jnp.sum(l * l, axis=1, keepdims=True))
    res = 20.0 * l / rn - cm
    if relu_out:
        res = jnp.maximum(res, 0.0)
    o_ref[...] = res


def _tc_tail(relu_out, h, p, Wa, ba, g, be, Wb, bb):
    return pl.pallas_call(
        functools.partial(_tc_tail_body, relu_out),
        out_shape=jax.ShapeDtypeStruct((N, H), jnp.float32),
    )(h, p, Wa, ba.reshape(1, H), g.reshape(1, H), be.reshape(1, H),
      Wb, bb.reshape(1, H))


def kernel(x, edge_index, W0, b0, W1a, b1a, g1, be1, W1b, b1b,
           W2a, b2a, g2, be2, W2b, b2b):
    pad = E_PER_W_PAD - E_PER_W
    src = jnp.pad(edge_index[0].reshape(NW, E_PER_W), ((0, 0), (0, pad)),
                  constant_values=0).reshape(NW, CHUNKS, CH)
    dst = jnp.pad(edge_index[1].reshape(NW, E_PER_W), ((0, 0), (0, pad)),
                  constant_values=N).reshape(NW, CHUNKS, CH)
    zeros = jnp.zeros((NP, D), jnp.float32)

    h0 = _tc_linear(x, W0, b0)
    p1 = _sc_segment_sum(h0, src, dst, zeros)
    l1 = _tc_tail(True, h0, p1, W1a, b1a, g1, be1, W1b, b1b)
    p2 = _sc_segment_sum(l1, src, dst, zeros)
    l2 = _tc_tail(False, l1, p2, W2a, b2a, g2, be2, W2b, b2b)
    return l2


# speedup vs baseline: 3.8665x; 1.1673x over previous
import functools

import jax
import jax.numpy as jnp
from jax import lax
from jax.experimental import pallas as pl
from jax.experimental.pallas import tpu as pltpu
from jax.experimental.pallas import tpu_sc as plsc

N = 10000
E = 320000
D = 128
H = 128

NC = 2
NS = 16
NW = NC * NS

NP = 10240
ROWS_PER_TILE = NP // NS
E_PER_W = E // NW
CH = 64
E_PER_W_PAD = 10240
CHUNKS = E_PER_W_PAD // CH

_sc_mesh = plsc.VectorSubcoreMesh(core_axis_name="c", subcore_axis_name="s")


@functools.partial(
    pl.kernel,
    out_type=jax.ShapeDtypeStruct((NC, NP, D), jnp.float32),
    mesh=_sc_mesh,
    scratch_types=[
        pltpu.VMEM((E_PER_W_PAD,), jnp.int32),
        pltpu.VMEM((CHUNKS, CH), jnp.int32),
        pltpu.VMEM((CH, D), jnp.float32),
        pltpu.VMEM((CH, D), jnp.float32),
        pltpu.VMEM_SHARED((NP, D), jnp.float32),
        pltpu.SemaphoreType.DMA,
        pltpu.SemaphoreType.DMA,
    ],
)
def _sc_segment_sum(h_hbm, src_hbm, dst_hbm, zeros_hbm, out_hbm,
                    src_v, dst_v, rows0_v, rows1_v, acc, sem0, sem1):
    c = lax.axis_index("c")
    s = lax.axis_index("s")
    wid = s * NC + c
    r0 = pl.multiple_of(s * ROWS_PER_TILE, ROWS_PER_TILE)
    pltpu.sync_copy(zeros_hbm.at[pl.ds(r0, ROWS_PER_TILE)],
                    acc.at[pl.ds(r0, ROWS_PER_TILE)])
    pltpu.sync_copy(src_hbm.at[wid], src_v)
    pltpu.sync_copy(dst_hbm.at[wid], dst_v)
    plsc.subcore_barrier()

    def _start_gather(j, buf, sem):
        return pltpu.async_copy(h_hbm.at[src_v.at[pl.ds(j * CH, CH)]],
                                buf, sem)

    _start_gather(0, rows0_v, sem0)
    _start_gather(1, rows1_v, sem1)

    def body(i, carry):
        j = pl.multiple_of(i * 2, 2)
        pltpu.make_async_copy(h_hbm.at[pl.ds(0, CH)], rows0_v, sem0).wait()
        pltpu.sync_copy(rows0_v, acc.at[dst_v.at[j]], add=True)

        @pl.when(j + 2 < CHUNKS)
        def _():
            _start_gather(j + 2, rows0_v, sem0)

        pltpu.make_async_copy(h_hbm.at[pl.ds(0, CH)], rows1_v, sem1).wait()
        pltpu.sync_copy(rows1_v, acc.at[dst_v.at[j + 1]], add=True)

        @pl.when(j + 3 < CHUNKS)
        def _():
            _start_gather(j + 3, rows1_v, sem1)

        return carry

    lax.fori_loop(0, CHUNKS // 2, body, 0, unroll=False)
    plsc.subcore_barrier()
    pltpu.sync_copy(acc.at[pl.ds(r0, ROWS_PER_TILE)],
                    out_hbm.at[c, pl.ds(r0, ROWS_PER_TILE)])


def _tc_linear_body(x_ref, w_ref, b_ref, o_ref):
    o_ref[...] = (jnp.dot(x_ref[...], w_ref[...],
                          preferred_element_type=jnp.float32) + b_ref[...])


def _tc_linear(x, W, b):
    return pl.pallas_call(
        _tc_linear_body,
        out_shape=jax.ShapeDtypeStruct((N, H), jnp.float32),
    )(x, W, b.reshape(1, H))


def _tc_tail_body(relu_out, h_ref, p_ref, wa_ref, ba_ref, g_ref, be_ref,
                  wb_ref, bb_ref, o_ref):
    out = h_ref[...] + p_ref[0, :N] + p_ref[1, :N]
    t = (jnp.dot(out, wa_ref[...], preferred_element_type=jnp.float32)
         + ba_ref[...])
    m = jnp.mean(t, axis=0, keepdims=True)
    v = jnp.mean((t - m) ** 2, axis=0, keepdims=True)
    t = (t - m) * lax.rsqrt(v + 1e-5) * g_ref[...] + be_ref[...]
    t = jnp.maximum(t, 0.0)
    l = (jnp.dot(t, wb_ref[...], preferred_element_type=jnp.float32)
         + bb_ref[...])
    cm = jnp.mean(l, axis=0, keepdims=True)
    rn = jnp.sqrt(1e-6 + jnp.sum(l * l, axis=1, keepdims=True))
    res = 20.0 * l / rn - cm
    if relu_out:
        res = jnp.maximum(res, 0.0)
    o_ref[...] = res


def _tc_tail(relu_out, h, p, Wa, ba, g, be, Wb, bb):
    return pl.pallas_call(
        functools.partial(_tc_tail_body, relu_out),
        out_shape=jax.ShapeDtypeStruct((N, H), jnp.float32),
    )(h, p, Wa, ba.reshape(1, H), g.reshape(1, H), be.reshape(1, H),
      Wb, bb.reshape(1, H))


def kernel(x, edge_index, W0, b0, W1a, b1a, g1, be1, W1b, b1b,
           W2a, b2a, g2, be2, W2b, b2b):
    pad = E_PER_W_PAD - E_PER_W
    src = jnp.pad(edge_index[0].reshape(NW, E_PER_W), ((0, 0), (0, pad)),
                  constant_values=0)
    dst = jnp.pad(edge_index[1].reshape(NW, E_PER_W), ((0, 0), (0, pad)),
                  constant_values=N).reshape(NW, CHUNKS, CH)
    zeros = jnp.zeros((NP, D), jnp.float32)

    h0 = _tc_linear(x, W0, b0)
    p1 = _sc_segment_sum(h0, src, dst, zeros)
    l1 = _tc_tail(True, h0, p1, W1a, b1a, g1, be1, W1b, b1b)
    p2 = _sc_segment_sum(l1, src, dst, zeros)
    l2 = _tc_tail(False, l1, p2, W2a, b2a, g2, be2, W2b, b2b)
    return l2
